# baseline (speedup 1.0000x reference)
import functools

import jax
import jax.numpy as jnp
from jax import lax
from jax.experimental import pallas as pl
from jax.experimental.pallas import tpu as pltpu
from jax.experimental.pallas import tpu_sc as plsc

E = 8
TOP_K = 2
D_MODEL = 2048
D_FF = 768

BM = 256
BR = 256


def _router_body(x_ref, wr_ref, a1_ref, a2_ref, w1_ref, w2_ref):
    xb = x_ref[...]
    wr = wr_ref[...]
    logits = jax.lax.dot_general(xb, wr, (((1,), (1,)), ((), ())),
                                 preferred_element_type=jnp.float32)
    col = jax.lax.broadcasted_iota(jnp.int32, logits.shape, 1)
    neg = jnp.float32(-jnp.inf)
    logits = jnp.where(col < E, logits, neg)
    m1 = jnp.max(logits, axis=1)
    a1 = jnp.argmax(logits, axis=1).astype(jnp.int32)
    masked = jnp.where(col == a1[:, None], neg, logits)
    m2 = jnp.max(masked, axis=1)
    a2 = jnp.argmax(masked, axis=1).astype(jnp.int32)
    w1 = jax.nn.sigmoid(m1 - m2)
    a1_ref[...] = a1
    a2_ref[...] = a2
    w1_ref[...] = w1
    w2_ref[...] = 1.0 - w1


def _router(x, Wr):
    T = x.shape[0]
    Wrp = jnp.zeros((128, D_MODEL), jnp.float32).at[:E].set(Wr)
    outs = pl.pallas_call(
        _router_body,
        grid=(T // BR,),
        in_specs=[
            pl.BlockSpec((BR, D_MODEL), lambda i: (i, 0)),
            pl.BlockSpec((128, D_MODEL), lambda i: (0, 0)),
        ],
        out_specs=[
            pl.BlockSpec((BR,), lambda i: (i,)),
            pl.BlockSpec((BR,), lambda i: (i,)),
            pl.BlockSpec((BR,), lambda i: (i,)),
            pl.BlockSpec((BR,), lambda i: (i,)),
        ],
        out_shape=[
            jax.ShapeDtypeStruct((T,), jnp.int32),
            jax.ShapeDtypeStruct((T,), jnp.int32),
            jax.ShapeDtypeStruct((T,), jnp.float32),
            jax.ShapeDtypeStruct((T,), jnp.float32),
        ],
    )(x, Wrp)
    return outs


def _mm_body(meta_ref, xs_ref, wg_ref, wu_ref, wd_ref, w_ref, ys_ref):
    i = pl.program_id(0)

    @pl.when(meta_ref[1, i] == 1)
    def _():
        xb = xs_ref[...]
        g = jax.lax.dot_general(xb, wg_ref[0], (((1,), (1,)), ((), ())),
                                preferred_element_type=jnp.float32)
        u = jax.lax.dot_general(xb, wu_ref[0], (((1,), (1,)), ((), ())),
                                preferred_element_type=jnp.float32)
        h = (g * jax.nn.sigmoid(g)) * u
        y = jax.lax.dot_general(h, wd_ref[0], (((1,), (1,)), ((), ())),
                                preferred_element_type=jnp.float32)
        ys_ref[...] = y * w_ref[:, :1]


def _grouped_mm(xs, Wg, Wu, Wd, w_padded, meta, nb):
    gp = xs.shape[0]
    w_bcast = jnp.broadcast_to(w_padded[:, None], (gp, 128))
    grid_spec = pltpu.PrefetchScalarGridSpec(
        num_scalar_prefetch=1,
        grid=(nb,),
        in_specs=[
            pl.BlockSpec((BM, D_MODEL), lambda i, m: (i, 0)),
            pl.BlockSpec((1, D_FF, D_MODEL), lambda i, m: (m[0, i], 0, 0)),
            pl.BlockSpec((1, D_FF, D_MODEL), lambda i, m: (m[0, i], 0, 0)),
            pl.BlockSpec((1, D_MODEL, D_FF), lambda i, m: (m[0, i], 0, 0)),
            pl.BlockSpec((BM, 128), lambda i, m: (i, 0)),
        ],
        out_specs=pl.BlockSpec((BM, D_MODEL), lambda i, m: (i, 0)),
    )
    return pl.pallas_call(
        _mm_body,
        grid_spec=grid_spec,
        out_shape=jax.ShapeDtypeStruct((gp, D_MODEL), jnp.float32),
    )(meta, xs, Wg, Wu, Wd, w_bcast)


def _dispatch(x, pa3, pb3, gp):
    T, D = x.shape
    NW = 32
    tpw = T // NW
    CH = 16
    nch = tpw // CH
    mesh = plsc.VectorSubcoreMesh(core_axis_name="c", subcore_axis_name="s")

    @functools.partial(
        pl.kernel, mesh=mesh,
        out_type=jax.ShapeDtypeStruct((gp, D), jnp.float32),
        scratch_types=[
            pltpu.VMEM((nch, CH), jnp.int32),
            pltpu.VMEM((nch, CH), jnp.int32),
            pltpu.VMEM((CH, D), jnp.float32),
            pltpu.SemaphoreType.DMA,
            pltpu.SemaphoreType.DMA,
        ],
    )
    def k(x_hbm, pa_hbm, pb_hbm, xs_hbm, pav, pbv, buf, sema, semb):
        wid = lax.axis_index("s") * 2 + lax.axis_index("c")
        base = wid * tpw
        pltpu.sync_copy(pa_hbm.at[wid], pav)
        pltpu.sync_copy(pb_hbm.at[wid], pbv)

        def chunk(c, carry):
            pltpu.sync_copy(x_hbm.at[pl.ds(base + c * CH, CH)], buf)
            ca = pltpu.async_copy(buf, xs_hbm.at[pav.at[c]], sema)
            cb = pltpu.async_copy(buf, xs_hbm.at[pbv.at[c]], semb)
            ca.wait()
            cb.wait()
            return carry

        lax.fori_loop(0, nch, chunk, 0)

    return k(x, pa3, pb3)


def _combine(ys, pint):
    F = pint.shape[0]
    T = F // 2
    D = ys.shape[1]
    NW = 32
    fpw = F // NW
    CH = 16
    nch = fpw // CH
    mesh = plsc.VectorSubcoreMesh(core_axis_name="c", subcore_axis_name="s")

    @functools.partial(
        pl.kernel, mesh=mesh,
        out_type=jax.ShapeDtypeStruct((T, D), jnp.float32),
        scratch_types=[
            pltpu.VMEM((fpw,), jnp.int32),
            pltpu.VMEM((CH, D), jnp.float32),
            pltpu.VMEM((CH, D), jnp.float32),
            pltpu.VMEM((CH // 2, D), jnp.float32),
            pltpu.VMEM((CH // 2, D), jnp.float32),
            pltpu.SemaphoreType.DMA,
            pltpu.SemaphoreType.DMA,
            pltpu.SemaphoreType.DMA,
            pltpu.SemaphoreType.DMA,
        ],
    )
    def k(ys_hbm, pint_hbm, out_hbm, p_v, buf0, buf1, ob0, ob1, sg0, sg1, so0, so1):
        wid = lax.axis_index("s") * 2 + lax.axis_index("c")
        fbase = wid * fpw
        tbase = wid * (fpw // 2)
        pltpu.sync_copy(pint_hbm.at[pl.ds(fbase, fpw)], p_v)
        bufs = [(buf0, ob0, sg0, so0), (buf1, ob1, sg1, so1)]

        def issue(c):
            buf, _, sg, _ = bufs[c % 2]
            return pltpu.async_copy(ys_hbm.at[p_v.at[pl.ds(c * CH, CH)]], buf, sg)

        pending = {0: issue(0)}
        stores = {}
        for c in range(nch):
            if c + 1 < nch:
                pending[c + 1] = issue(c + 1)
            pending.pop(c).wait()
            buf, ob, _, so = bufs[c % 2]
            if c - 2 in stores:
                stores.pop(c - 2).wait()

            def row(r, carry2, buf=buf, ob=ob):
                for j in range(D // 16):
                    sl = pl.ds(j * 16, 16)
                    ob[r, sl] = buf[2 * r, sl] + buf[2 * r + 1, sl]
                return carry2

            lax.fori_loop(0, CH // 2, row, 0)
            stores[c] = pltpu.async_copy(
                ob, out_hbm.at[pl.ds(tbase + c * (CH // 2), CH // 2)], so)
        for st in stores.values():
            st.wait()

    return k(ys, pint)


def kernel(hidden_states, Wr, Wg, Wu, Wd):
    b, s, d = hidden_states.shape
    T = b * s
    nb = T * TOP_K // BM + E
    gp = nb * BM
    x = hidden_states.reshape(T, d)

    a1, a2, w1, w2 = _router(x, Wr)

    se_flat = jnp.stack([a1, a2], axis=-1).reshape(-1)
    w_flat = jnp.stack([w1, w2], axis=-1).reshape(-1)
    tok_flat = jnp.arange(2 * T, dtype=jnp.int32) // 2
    onehot = (se_flat[:, None] == jnp.arange(E, dtype=jnp.int32)[None, :]).astype(jnp.int32)
    counts = jnp.sum(onehot, axis=0)
    rank_within = jnp.sum((jnp.cumsum(onehot, axis=0) - onehot) * onehot, axis=1)
    blocks_per_e = (counts + BM - 1) // BM
    pad_off = BM * (jnp.cumsum(blocks_per_e) - blocks_per_e)
    p_of_flat = pad_off[se_flat] + rank_within
    w_padded = jnp.zeros((gp,), jnp.float32).at[p_of_flat].set(w_flat)
    q = jnp.arange(nb, dtype=jnp.int32) * BM
    eid = jnp.sum((q[:, None] >= pad_off[None, :]).astype(jnp.int32), axis=-1) - 1
    eid = jnp.clip(eid, 0, E - 1)
    active = (q < (pad_off + BM * blocks_per_e)[eid]).astype(jnp.int32)
    meta = jnp.stack([eid, active])

    pa = p_of_flat[0::2]
    pb = p_of_flat[1::2]
    xs = _dispatch(x, pa.reshape(32, -1, 16), pb.reshape(32, -1, 16), gp)
    ys = _grouped_mm(xs, Wg, Wu, Wd, w_padded, meta, nb)
    out = _combine(ys, p_of_flat)
    return out.reshape(b, s, d)


# speedup vs baseline: 1.0391x; 1.0391x over previous
import functools

import jax
import jax.numpy as jnp
from jax import lax
from jax.experimental import pallas as pl
from jax.experimental.pallas import tpu as pltpu
from jax.experimental.pallas import tpu_sc as plsc

E = 8
TOP_K = 2
D_MODEL = 2048
D_FF = 768

BM = 256
BR = 256


def _router_body(x_ref, wr_ref, a1_ref, a2_ref, w1_ref, w2_ref,
                 r1_ref, r2_ref, cnt_ref, run_ref):
    i = pl.program_id(0)

    @pl.when(i == 0)
    def _():
        run_ref[...] = jnp.zeros_like(run_ref)

    xb = x_ref[...]
    wr = wr_ref[...]
    logits = jax.lax.dot_general(xb, wr, (((1,), (1,)), ((), ())),
                                 preferred_element_type=jnp.float32)
    col = jax.lax.broadcasted_iota(jnp.int32, logits.shape, 1)
    neg = jnp.float32(-jnp.inf)
    logits = jnp.where(col < E, logits, neg)
    m1 = jnp.max(logits, axis=1)
    a1 = jnp.argmax(logits, axis=1).astype(jnp.int32)
    masked = jnp.where(col == a1[:, None], neg, logits)
    m2 = jnp.max(masked, axis=1)
    a2 = jnp.argmax(masked, axis=1).astype(jnp.int32)
    w1 = jax.nn.sigmoid(m1 - m2)
    a1_ref[...] = a1
    a2_ref[...] = a2
    w1_ref[...] = w1
    w2_ref[...] = 1.0 - w1
    oh1 = (col == a1[:, None]).astype(jnp.float32)
    oh2 = (col == a2[:, None]).astype(jnp.float32)
    both = oh1 + oh2
    r = jax.lax.broadcasted_iota(jnp.int32, (both.shape[0], both.shape[0]), 0)
    c = jax.lax.broadcasted_iota(jnp.int32, (both.shape[0], both.shape[0]), 1)
    tri = (c < r).astype(jnp.float32)
    cum = jax.lax.dot_general(tri, both, (((1,), (0,)), ((), ())),
                              preferred_element_type=jnp.float32)
    pos = cum + run_ref[...]
    r1_ref[...] = jnp.sum(oh1 * pos, axis=1).astype(jnp.int32)
    r2_ref[...] = jnp.sum(oh2 * pos, axis=1).astype(jnp.int32)
    new_run = run_ref[...] + jnp.sum(both, axis=0, keepdims=True)
    run_ref[...] = new_run
    cnt_ref[...] = new_run


def _router(x, Wr):
    T = x.shape[0]
    Wrp = jnp.zeros((128, D_MODEL), jnp.float32).at[:E].set(Wr)
    outs = pl.pallas_call(
        _router_body,
        grid=(T // BR,),
        in_specs=[
            pl.BlockSpec((BR, D_MODEL), lambda i: (i, 0)),
            pl.BlockSpec((128, D_MODEL), lambda i: (0, 0)),
        ],
        out_specs=[
            pl.BlockSpec((BR,), lambda i: (i,)),
            pl.BlockSpec((BR,), lambda i: (i,)),
            pl.BlockSpec((BR,), lambda i: (i,)),
            pl.BlockSpec((BR,), lambda i: (i,)),
            pl.BlockSpec((BR,), lambda i: (i,)),
            pl.BlockSpec((BR,), lambda i: (i,)),
            pl.BlockSpec((1, 128), lambda i: (0, 0)),
        ],
        out_shape=[
            jax.ShapeDtypeStruct((T,), jnp.int32),
            jax.ShapeDtypeStruct((T,), jnp.int32),
            jax.ShapeDtypeStruct((T,), jnp.float32),
            jax.ShapeDtypeStruct((T,), jnp.float32),
            jax.ShapeDtypeStruct((T,), jnp.int32),
            jax.ShapeDtypeStruct((T,), jnp.int32),
            jax.ShapeDtypeStruct((1, 128), jnp.float32),
        ],
        scratch_shapes=[pltpu.VMEM((1, 128), jnp.float32)],
    )(x, Wrp)
    return outs


def _mm_body(meta_ref, xs_ref, wg_ref, wu_ref, wd_ref, w_ref, ys_ref):
    i = pl.program_id(0)

    @pl.when(meta_ref[1, i] == 1)
    def _():
        xb = xs_ref[...]
        g = jax.lax.dot_general(xb, wg_ref[0], (((1,), (1,)), ((), ())),
                                preferred_element_type=jnp.float32)
        u = jax.lax.dot_general(xb, wu_ref[0], (((1,), (1,)), ((), ())),
                                preferred_element_type=jnp.float32)
        h = (g * jax.nn.sigmoid(g)) * u
        y = jax.lax.dot_general(h, wd_ref[0], (((1,), (1,)), ((), ())),
                                preferred_element_type=jnp.float32)
        ys_ref[...] = y * w_ref[:, :1]


def _grouped_mm(xs, Wg, Wu, Wd, w_padded, meta, nb):
    gp = xs.shape[0]
    w_bcast = jnp.broadcast_to(w_padded[:, None], (gp, 128))
    grid_spec = pltpu.PrefetchScalarGridSpec(
        num_scalar_prefetch=1,
        grid=(nb,),
        in_specs=[
            pl.BlockSpec((BM, D_MODEL), lambda i, m: (i, 0)),
            pl.BlockSpec((1, D_FF, D_MODEL), lambda i, m: (m[0, i], 0, 0)),
            pl.BlockSpec((1, D_FF, D_MODEL), lambda i, m: (m[0, i], 0, 0)),
            pl.BlockSpec((1, D_MODEL, D_FF), lambda i, m: (m[0, i], 0, 0)),
            pl.BlockSpec((BM, 128), lambda i, m: (i, 0)),
        ],
        out_specs=pl.BlockSpec((BM, D_MODEL), lambda i, m: (i, 0)),
    )
    return pl.pallas_call(
        _mm_body,
        grid_spec=grid_spec,
        out_shape=jax.ShapeDtypeStruct((gp, D_MODEL), jnp.float32),
    )(meta, xs, Wg, Wu, Wd, w_bcast)


def _dispatch(x, pa3, pb3, gp):
    T, D = x.shape
    NW = 32
    tpw = T // NW
    CH = 16
    nch = tpw // CH
    mesh = plsc.VectorSubcoreMesh(core_axis_name="c", subcore_axis_name="s")

    @functools.partial(
        pl.kernel, mesh=mesh,
        out_type=jax.ShapeDtypeStruct((gp, D), jnp.float32),
        scratch_types=[
            pltpu.VMEM((nch, CH), jnp.int32),
            pltpu.VMEM((nch, CH), jnp.int32),
            pltpu.VMEM((CH, D), jnp.float32),
            pltpu.SemaphoreType.DMA,
            pltpu.SemaphoreType.DMA,
        ],
    )
    def k(x_hbm, pa_hbm, pb_hbm, xs_hbm, pav, pbv, buf, sema, semb):
        wid = lax.axis_index("s") * 2 + lax.axis_index("c")
        base = wid * tpw
        pltpu.sync_copy(pa_hbm.at[wid], pav)
        pltpu.sync_copy(pb_hbm.at[wid], pbv)

        def chunk(c, carry):
            pltpu.sync_copy(x_hbm.at[pl.ds(base + c * CH, CH)], buf)
            ca = pltpu.async_copy(buf, xs_hbm.at[pav.at[c]], sema)
            cb = pltpu.async_copy(buf, xs_hbm.at[pbv.at[c]], semb)
            ca.wait()
            cb.wait()
            return carry

        lax.fori_loop(0, nch, chunk, 0)

    return k(x, pa3, pb3)


def _combine(ys, pa, pb):
    T = pa.shape[0]
    D = ys.shape[1]
    NW = 32
    tpw = T // NW
    CH = 8
    nch = tpw // CH
    mesh = plsc.VectorSubcoreMesh(core_axis_name="c", subcore_axis_name="s")

    @functools.partial(
        pl.kernel, mesh=mesh,
        out_type=jax.ShapeDtypeStruct((T, D), jnp.float32),
        scratch_types=[
            pltpu.VMEM((tpw,), jnp.int32),
            pltpu.VMEM((tpw,), jnp.int32),
            pltpu.VMEM((CH, D), jnp.float32),
            pltpu.VMEM((CH, D), jnp.float32),
            pltpu.VMEM((CH, D), jnp.float32),
            pltpu.VMEM((CH, D), jnp.float32),
            pltpu.SemaphoreType.DMA,
            pltpu.SemaphoreType.DMA,
            pltpu.SemaphoreType.DMA,
            pltpu.SemaphoreType.DMA,
        ],
    )
    def k(ys_hbm, pa_hbm, pb_hbm, out_hbm, pa_v, pb_v,
          bufa0, bufb0, bufa1, bufb1, sa0, sb0, sa1, sb1):
        wid = lax.axis_index("s") * 2 + lax.axis_index("c")
        base = wid * tpw
        pltpu.sync_copy(pa_hbm.at[pl.ds(base, tpw)], pa_v)
        pltpu.sync_copy(pb_hbm.at[pl.ds(base, tpw)], pb_v)
        bufs = [(bufa0, bufb0, sa0, sb0), (bufa1, bufb1, sa1, sb1)]

        def issue(c):
            ba, bb, sa, sb = bufs[c % 2]
            sl = pl.ds(c * CH, CH)
            ca = pltpu.async_copy(ys_hbm.at[pa_v.at[sl]], ba, sa)
            cb = pltpu.async_copy(ys_hbm.at[pb_v.at[sl]], bb, sb)
            return ca, cb

        pending = {0: issue(0)}
        for c in range(nch):
            if c + 1 < nch:
                pending[c + 1] = issue(c + 1)
            ca, cb = pending.pop(c)
            ca.wait()
            cb.wait()
            ba, bb, _, _ = bufs[c % 2]

            def row(r, carry2, ba=ba, bb=bb):
                for j in range(D // 16):
                    sl = pl.ds(j * 16, 16)
                    ba[r, sl] = ba[r, sl] + bb[r, sl]
                return carry2

            lax.fori_loop(0, CH, row, 0)
            pltpu.sync_copy(ba, out_hbm.at[pl.ds(base + c * CH, CH)])

    return k(ys, pa, pb)


def kernel(hidden_states, Wr, Wg, Wu, Wd):
    b, s, d = hidden_states.shape
    T = b * s
    nb = T * TOP_K // BM + E
    gp = nb * BM
    x = hidden_states.reshape(T, d)

    a1, a2, w1, w2, r1, r2, cnt = _router(x, Wr)

    se_flat = jnp.stack([a1, a2], axis=-1).reshape(-1)
    w_flat = jnp.stack([w1, w2], axis=-1).reshape(-1)
    rank_flat = jnp.stack([r1, r2], axis=-1).reshape(-1)
    counts = cnt[0, :E].astype(jnp.int32)
    blocks_per_e = (counts + BM - 1) // BM
    pad_off = BM * (jnp.cumsum(blocks_per_e) - blocks_per_e)
    p_of_flat = pad_off[se_flat] + rank_flat
    w_padded = jnp.zeros((gp,), jnp.float32).at[p_of_flat].set(w_flat)
    q = jnp.arange(nb, dtype=jnp.int32) * BM
    eid = jnp.sum((q[:, None] >= pad_off[None, :]).astype(jnp.int32), axis=-1) - 1
    eid = jnp.clip(eid, 0, E - 1)
    active = (q < (pad_off + BM * blocks_per_e)[eid]).astype(jnp.int32)
    meta = jnp.stack([eid, active])

    pa = p_of_flat[0::2]
    pb = p_of_flat[1::2]
    xs = _dispatch(x, pa.reshape(32, -1, 16), pb.reshape(32, -1, 16), gp)
    ys = _grouped_mm(xs, Wg, Wu, Wd, w_padded, meta, nb)
    out = _combine(ys, pa, pb)
    return out.reshape(b, s, d)


# speedup vs baseline: 1.0484x; 1.0090x over previous
import functools

import jax
import jax.numpy as jnp
from jax import lax
from jax.experimental import pallas as pl
from jax.experimental.pallas import tpu as pltpu
from jax.experimental.pallas import tpu_sc as plsc

E = 8
TOP_K = 2
D_MODEL = 2048
D_FF = 768

BM = 256
BR = 256


def _router_body(x_ref, wr_ref, a1_ref, a2_ref, w1_ref, w2_ref,
                 r1_ref, r2_ref, cnt_ref, run_ref):
    i = pl.program_id(0)

    @pl.when(i == 0)
    def _():
        run_ref[...] = jnp.zeros_like(run_ref)

    xb = x_ref[...]
    wr = wr_ref[...]
    logits = jax.lax.dot_general(xb, wr, (((1,), (1,)), ((), ())),
                                 preferred_element_type=jnp.float32)
    col = jax.lax.broadcasted_iota(jnp.int32, logits.shape, 1)
    neg = jnp.float32(-jnp.inf)
    logits = jnp.where(col < E, logits, neg)
    m1 = jnp.max(logits, axis=1)
    a1 = jnp.argmax(logits, axis=1).astype(jnp.int32)
    masked = jnp.where(col == a1[:, None], neg, logits)
    m2 = jnp.max(masked, axis=1)
    a2 = jnp.argmax(masked, axis=1).astype(jnp.int32)
    w1 = jax.nn.sigmoid(m1 - m2)
    a1_ref[...] = a1
    a2_ref[...] = a2
    w1_ref[...] = w1
    w2_ref[...] = 1.0 - w1
    oh1 = (col == a1[:, None]).astype(jnp.float32)
    oh2 = (col == a2[:, None]).astype(jnp.float32)
    both = oh1 + oh2
    r = jax.lax.broadcasted_iota(jnp.int32, (both.shape[0], both.shape[0]), 0)
    c = jax.lax.broadcasted_iota(jnp.int32, (both.shape[0], both.shape[0]), 1)
    tri = (c < r).astype(jnp.float32)
    cum = jax.lax.dot_general(tri, both, (((1,), (0,)), ((), ())),
                              preferred_element_type=jnp.float32)
    pos = cum + run_ref[...]
    r1_ref[...] = jnp.sum(oh1 * pos, axis=1).astype(jnp.int32)
    r2_ref[...] = jnp.sum(oh2 * pos, axis=1).astype(jnp.int32)
    new_run = run_ref[...] + jnp.sum(both, axis=0, keepdims=True)
    run_ref[...] = new_run
    cnt_ref[...] = new_run


def _router(x, Wr):
    T = x.shape[0]
    Wrp = jnp.zeros((128, D_MODEL), jnp.float32).at[:E].set(Wr)
    outs = pl.pallas_call(
        _router_body,
        grid=(T // BR,),
        in_specs=[
            pl.BlockSpec((BR, D_MODEL), lambda i: (i, 0)),
            pl.BlockSpec((128, D_MODEL), lambda i: (0, 0)),
        ],
        out_specs=[
            pl.BlockSpec((BR,), lambda i: (i,)),
            pl.BlockSpec((BR,), lambda i: (i,)),
            pl.BlockSpec((BR,), lambda i: (i,)),
            pl.BlockSpec((BR,), lambda i: (i,)),
            pl.BlockSpec((BR,), lambda i: (i,)),
            pl.BlockSpec((BR,), lambda i: (i,)),
            pl.BlockSpec((1, 128), lambda i: (0, 0)),
        ],
        out_shape=[
            jax.ShapeDtypeStruct((T,), jnp.int32),
            jax.ShapeDtypeStruct((T,), jnp.int32),
            jax.ShapeDtypeStruct((T,), jnp.float32),
            jax.ShapeDtypeStruct((T,), jnp.float32),
            jax.ShapeDtypeStruct((T,), jnp.int32),
            jax.ShapeDtypeStruct((T,), jnp.int32),
            jax.ShapeDtypeStruct((1, 128), jnp.float32),
        ],
        scratch_shapes=[pltpu.VMEM((1, 128), jnp.float32)],
    )(x, Wrp)
    return outs


def _mm_body(meta_ref, xs_ref, wg_ref, wu_ref, wd_ref, w_ref, ys_ref):
    i = pl.program_id(0)

    @pl.when(meta_ref[1, i] == 1)
    def _():
        xb = xs_ref[...]
        g = jax.lax.dot_general(xb, wg_ref[0], (((1,), (1,)), ((), ())),
                                preferred_element_type=jnp.float32)
        u = jax.lax.dot_general(xb, wu_ref[0], (((1,), (1,)), ((), ())),
                                preferred_element_type=jnp.float32)
        h = (g * jax.nn.sigmoid(g)) * u
        y = jax.lax.dot_general(h, wd_ref[0], (((1,), (1,)), ((), ())),
                                preferred_element_type=jnp.float32)
        ys_ref[...] = y * w_ref[:, :1]


def _grouped_mm(xs, Wg, Wu, Wd, w_padded, meta, nb):
    gp = xs.shape[0]
    w_bcast = jnp.broadcast_to(w_padded[:, None], (gp, 128))
    grid_spec = pltpu.PrefetchScalarGridSpec(
        num_scalar_prefetch=1,
        grid=(nb,),
        in_specs=[
            pl.BlockSpec((BM, D_MODEL), lambda i, m: (i, 0)),
            pl.BlockSpec((1, D_FF, D_MODEL), lambda i, m: (m[0, i], 0, 0)),
            pl.BlockSpec((1, D_FF, D_MODEL), lambda i, m: (m[0, i], 0, 0)),
            pl.BlockSpec((1, D_MODEL, D_FF), lambda i, m: (m[0, i], 0, 0)),
            pl.BlockSpec((BM, 128), lambda i, m: (i, 0)),
        ],
        out_specs=pl.BlockSpec((BM, D_MODEL), lambda i, m: (i, 0)),
    )
    return pl.pallas_call(
        _mm_body,
        grid_spec=grid_spec,
        out_shape=jax.ShapeDtypeStruct((gp, D_MODEL), jnp.float32),
    )(meta, xs, Wg, Wu, Wd, w_bcast)


def _dispatch(x, pa3, pb3, gp):
    T, D = x.shape
    NW = 32
    tpw = T // NW
    CH = 16
    nch = tpw // CH
    mesh = plsc.VectorSubcoreMesh(core_axis_name="c", subcore_axis_name="s")

    @functools.partial(
        pl.kernel, mesh=mesh,
        out_type=jax.ShapeDtypeStruct((gp, D), jnp.float32),
        scratch_types=[
            pltpu.VMEM((nch, CH), jnp.int32),
            pltpu.VMEM((nch, CH), jnp.int32),
            pltpu.VMEM((CH, D), jnp.float32),
            pltpu.VMEM((CH, D), jnp.float32),
            pltpu.SemaphoreType.DMA,
            pltpu.SemaphoreType.DMA,
            pltpu.SemaphoreType.DMA,
            pltpu.SemaphoreType.DMA,
            pltpu.SemaphoreType.DMA,
            pltpu.SemaphoreType.DMA,
        ],
    )
    def k(x_hbm, pa_hbm, pb_hbm, xs_hbm, pav, pbv,
          buf0, buf1, sr0, sr1, sa0, sb0, sa1, sb1):
        wid = lax.axis_index("s") * 2 + lax.axis_index("c")
        base = wid * tpw
        pltpu.sync_copy(pa_hbm.at[wid], pav)
        pltpu.sync_copy(pb_hbm.at[wid], pbv)
        bufs = [(buf0, sr0, sa0, sb0), (buf1, sr1, sa1, sb1)]

        def read(c):
            buf, sr, _, _ = bufs[c % 2]
            return pltpu.async_copy(x_hbm.at[pl.ds(base + c * CH, CH)], buf, sr)

        reads = {0: read(0)}
        scats = {}
        for c in range(nch):
            reads.pop(c).wait()
            if c + 1 < nch:
                if c - 1 in scats:
                    for cp in scats.pop(c - 1):
                        cp.wait()
                reads[c + 1] = read(c + 1)
            buf, _, sa, sb = bufs[c % 2]
            ca = pltpu.async_copy(buf, xs_hbm.at[pav.at[c]], sa)
            cb = pltpu.async_copy(buf, xs_hbm.at[pbv.at[c]], sb)
            scats[c] = (ca, cb)
        for cps in scats.values():
            for cp in cps:
                cp.wait()

    return k(x, pa3, pb3)


def _combine(ys, pa, pb):
    T = pa.shape[0]
    D = ys.shape[1]
    NW = 32
    tpw = T // NW
    CH = 8
    nch = tpw // CH
    mesh = plsc.VectorSubcoreMesh(core_axis_name="c", subcore_axis_name="s")

    @functools.partial(
        pl.kernel, mesh=mesh,
        out_type=jax.ShapeDtypeStruct((T, D), jnp.float32),
        scratch_types=[
            pltpu.VMEM((tpw,), jnp.int32),
            pltpu.VMEM((tpw,), jnp.int32),
            pltpu.VMEM((CH, D), jnp.float32),
            pltpu.VMEM((CH, D), jnp.float32),
            pltpu.VMEM((CH, D), jnp.float32),
            pltpu.VMEM((CH, D), jnp.float32),
            pltpu.SemaphoreType.DMA,
            pltpu.SemaphoreType.DMA,
            pltpu.SemaphoreType.DMA,
            pltpu.SemaphoreType.DMA,
        ],
    )
    def k(ys_hbm, pa_hbm, pb_hbm, out_hbm, pa_v, pb_v,
          bufa0, bufb0, bufa1, bufb1, sa0, sb0, sa1, sb1):
        wid = lax.axis_index("s") * 2 + lax.axis_index("c")
        base = wid * tpw
        pltpu.sync_copy(pa_hbm.at[pl.ds(base, tpw)], pa_v)
        pltpu.sync_copy(pb_hbm.at[pl.ds(base, tpw)], pb_v)
        bufs = [(bufa0, bufb0, sa0, sb0), (bufa1, bufb1, sa1, sb1)]

        def issue(c):
            ba, bb, sa, sb = bufs[c % 2]
            sl = pl.ds(c * CH, CH)
            ca = pltpu.async_copy(ys_hbm.at[pa_v.at[sl]], ba, sa)
            cb = pltpu.async_copy(ys_hbm.at[pb_v.at[sl]], bb, sb)
            return ca, cb

        pending = {0: issue(0)}
        for c in range(nch):
            if c + 1 < nch:
                pending[c + 1] = issue(c + 1)
            ca, cb = pending.pop(c)
            ca.wait()
            cb.wait()
            ba, bb, _, _ = bufs[c % 2]

            def row(r, carry2, ba=ba, bb=bb):
                for j in range(D // 16):
                    sl = pl.ds(j * 16, 16)
                    ba[r, sl] = ba[r, sl] + bb[r, sl]
                return carry2

            lax.fori_loop(0, CH, row, 0)
            pltpu.sync_copy(ba, out_hbm.at[pl.ds(base + c * CH, CH)])

    return k(ys, pa, pb)


def kernel(hidden_states, Wr, Wg, Wu, Wd):
    b, s, d = hidden_states.shape
    T = b * s
    nb = T * TOP_K // BM + E
    gp = nb * BM
    x = hidden_states.reshape(T, d)

    a1, a2, w1, w2, r1, r2, cnt = _router(x, Wr)

    se_flat = jnp.stack([a1, a2], axis=-1).reshape(-1)
    w_flat = jnp.stack([w1, w2], axis=-1).reshape(-1)
    rank_flat = jnp.stack([r1, r2], axis=-1).reshape(-1)
    counts = cnt[0, :E].astype(jnp.int32)
    blocks_per_e = (counts + BM - 1) // BM
    pad_off = BM * (jnp.cumsum(blocks_per_e) - blocks_per_e)
    p_of_flat = pad_off[se_flat] + rank_flat
    w_padded = jnp.zeros((gp,), jnp.float32).at[p_of_flat].set(w_flat)
    q = jnp.arange(nb, dtype=jnp.int32) * BM
    eid = jnp.sum((q[:, None] >= pad_off[None, :]).astype(jnp.int32), axis=-1) - 1
    eid = jnp.clip(eid, 0, E - 1)
    active = (q < (pad_off + BM * blocks_per_e)[eid]).astype(jnp.int32)
    meta = jnp.stack([eid, active])

    pa = p_of_flat[0::2]
    pb = p_of_flat[1::2]
    xs = _dispatch(x, pa.reshape(32, -1, 16), pb.reshape(32, -1, 16), gp)
    ys = _grouped_mm(xs, Wg, Wu, Wd, w_padded, meta, nb)
    out = _combine(ys, pa, pb)
    return out.reshape(b, s, d)
